# baseline (speedup 1.0000x reference)
import dataclasses
import functools

import jax
import jax.numpy as jnp
from jax import lax
from jax.experimental import pallas as pl
from jax.experimental.pallas import tpu as pltpu
from jax.experimental.pallas import tpu_sc as plsc

_UNROLLED_ITERS = 6
_MAX_EXTRA_ITERS = 26


def _sparsemax_block(x_ref, o_ref):
    x = x_ref[...]

    def newton(tau):
        mask = x > tau
        g = jnp.where(mask, x - tau, 0.0)
        s = jnp.sum(g, axis=-1, keepdims=True)
        c = jnp.sum(mask.astype(jnp.float32), axis=-1, keepdims=True)
        return tau + (s - 1.0) / c

    tau = jnp.max(x, axis=-1, keepdims=True) - 1.0
    for _ in range(_UNROLLED_ITERS):
        tau = newton(tau)

    def cond(carry):
        k, _, changed = carry
        return jnp.logical_and(k < _MAX_EXTRA_ITERS, changed)

    def body(carry):
        k, tau, _ = carry
        tau_new = newton(tau)
        return k + 1, tau_new, jnp.any(tau_new != tau)

    _, tau, _ = jax.lax.while_loop(cond, body, (0, tau, jnp.bool_(True)))
    o_ref[...] = jnp.maximum(x - tau, 0.0)


_ROW_BLOCK = 8


_SC_LANES = 16
_SC_CORES = 2
_SC_SUBCORES = 16
_SC_WORKERS = _SC_CORES * _SC_SUBCORES


def _sc_sdiv(a, b):
    q = jnp.full((_SC_LANES,), a) / jnp.full((_SC_LANES,), b)
    return jnp.max(q)


def _sc_row_sparsemax(row_v, idx_ref, n_chunks):
    L = _SC_LANES
    zeros = jnp.zeros((L,), jnp.float32)

    def max_body(i, acc):
        return jnp.maximum(acc, row_v[pl.ds(i * L, L)])

    acc = lax.fori_loop(1, n_chunks, max_body, row_v[pl.ds(0, L)])
    tau0 = jnp.max(acc) - 1.0

    def screen_body(i, carry):
        s, c, nch = carry
        v = row_v[pl.ds(i * L, L)]
        mask = v > tau0
        s = s + jnp.where(mask, v - tau0, 0.0)
        c = c + jnp.where(mask, 1.0, 0.0)

        def append(n):
            idx_ref[n] = i
            return n + 1

        nch = lax.cond(jnp.any(mask), append, lambda n: n, nch)
        return s, c, nch

    s16, c16, nch = lax.fori_loop(0, n_chunks, screen_body, (zeros, zeros, 0))
    tau = tau0 + _sc_sdiv(jnp.sum(s16) - 1.0, jnp.sum(c16))

    def newton(tau):
        def nb(j, carry):
            s, c = carry
            v = row_v[pl.ds(idx_ref[j] * L, L)]
            mask = v > tau
            s = s + jnp.where(mask, v - tau, 0.0)
            c = c + jnp.where(mask, 1.0, 0.0)
            return s, c

        s16, c16 = lax.fori_loop(0, nch, nb, (zeros, zeros))
        return tau + _sc_sdiv(jnp.sum(s16) - 1.0, jnp.sum(c16))

    def w_cond(carry):
        k, _, changed = carry
        return jnp.logical_and(k < 30, changed)

    def w_body(carry):
        k, tau, _ = carry
        t2 = newton(tau)
        return k + 1, t2, t2 != tau

    _, tau, _ = lax.while_loop(w_cond, w_body, (0, tau, jnp.bool_(True)))

    def out_body(i, carry):
        sl = pl.ds(i * L, L)
        row_v[sl] = jnp.maximum(row_v[sl] - tau, 0.0)
        return carry

    lax.fori_loop(0, n_chunks, out_body, 0)


def _sc_sparsemax(x):
    rows, cols = x.shape
    n_chunks = cols // _SC_LANES
    rpw = rows // _SC_WORKERS
    mesh = plsc.VectorSubcoreMesh(core_axis_name="c", subcore_axis_name="s")
    cp = pltpu.CompilerParams()
    if "needs_layout_passes" in pltpu.CompilerParams.__dataclass_fields__:
        cp = dataclasses.replace(cp, needs_layout_passes=False)

    @functools.partial(
        pl.kernel,
        out_type=jax.ShapeDtypeStruct((rows, cols), x.dtype),
        mesh=mesh,
        compiler_params=cp,
        scratch_types=[
            pltpu.VMEM((cols,), jnp.float32),
            pltpu.SMEM((n_chunks,), jnp.int32),
        ],
    )
    def k(x_hbm, o_hbm, row_v, idx_ref):
        wid = lax.axis_index("s") * _SC_CORES + lax.axis_index("c")
        for r in range(rpw):
            row = wid * rpw + r
            pltpu.sync_copy(x_hbm.at[row], row_v)
            _sc_row_sparsemax(row_v, idx_ref, n_chunks)
            pltpu.sync_copy(row_v, o_hbm.at[row])

    return k(x)


@functools.partial(jax.jit, static_argnames=())
def kernel(x):
    return _sc_sparsemax(x)


def _tc_kernel(x):
    return pl.pallas_call(
        _sparsemax_block,
        out_shape=jax.ShapeDtypeStruct(x.shape, x.dtype),
    )(x)


# speedup vs baseline: 5.8309x; 5.8309x over previous
import dataclasses
import functools

import jax
import jax.numpy as jnp
from jax import lax
from jax.experimental import pallas as pl
from jax.experimental.pallas import tpu as pltpu
from jax.experimental.pallas import tpu_sc as plsc

_UNROLLED_ITERS = 7
_MAX_EXTRA_ITERS = 24


def _sparsemax_block(x_ref, o_ref):
    x = x_ref[...]

    def feval(t):
        return jnp.sum(jnp.maximum(x - t, 0.0), axis=-1, keepdims=True) - 1.0

    def secant(t0, f0, t1, f1):
        denom = f1 - f0
        return jnp.where(denom < 0.0, t1 - f1 * (t1 - t0) / denom, t1)

    m = jnp.max(x, axis=-1, keepdims=True)
    t0 = m - 2.0
    f0 = feval(t0)
    t1 = m - 1.0
    f1 = feval(t1)
    for _ in range(_UNROLLED_ITERS):
        t2 = secant(t0, f0, t1, f1)
        t0, f0, t1, f1 = t1, f1, t2, feval(t2)

    def cond(carry):
        k = carry[0]
        changed = carry[5]
        return jnp.logical_and(k < _MAX_EXTRA_ITERS, changed)

    def body(carry):
        k, t0, f0, t1, f1, _ = carry
        t2 = secant(t0, f0, t1, f1)
        return k + 1, t1, f1, t2, feval(t2), jnp.any(t2 != t1)

    _, _, _, t1, _, _ = jax.lax.while_loop(
        cond, body, (0, t0, f0, t1, f1, jnp.bool_(True))
    )
    o_ref[...] = jnp.maximum(x - t1, 0.0)


_ROW_BLOCK = 8


_SC_LANES = 16
_SC_CORES = 2
_SC_SUBCORES = 16
_SC_WORKERS = _SC_CORES * _SC_SUBCORES


def _sc_sdiv(a, b):
    q = jnp.full((_SC_LANES,), a) / jnp.full((_SC_LANES,), b)
    return jnp.max(q)


def _sc_row_sparsemax(row_v, idx_ref, n_chunks):
    L = _SC_LANES
    zeros = jnp.zeros((L,), jnp.float32)

    def max_body(i, acc):
        return jnp.maximum(acc, row_v[pl.ds(i * L, L)])

    acc = lax.fori_loop(1, n_chunks, max_body, row_v[pl.ds(0, L)])
    tau0 = jnp.max(acc) - 1.0

    def screen_body(i, carry):
        s, c, nch = carry
        v = row_v[pl.ds(i * L, L)]
        mask = v > tau0
        s = s + jnp.where(mask, v - tau0, 0.0)
        c = c + jnp.where(mask, 1.0, 0.0)

        def append(n):
            idx_ref[n] = i
            return n + 1

        nch = lax.cond(jnp.any(mask), append, lambda n: n, nch)
        return s, c, nch

    s16, c16, nch = lax.fori_loop(0, n_chunks, screen_body, (zeros, zeros, 0))
    tau = tau0 + _sc_sdiv(jnp.sum(s16) - 1.0, jnp.sum(c16))

    def newton(tau):
        def nb(j, carry):
            s, c = carry
            v = row_v[pl.ds(idx_ref[j] * L, L)]
            mask = v > tau
            s = s + jnp.where(mask, v - tau, 0.0)
            c = c + jnp.where(mask, 1.0, 0.0)
            return s, c

        s16, c16 = lax.fori_loop(0, nch, nb, (zeros, zeros))
        return tau + _sc_sdiv(jnp.sum(s16) - 1.0, jnp.sum(c16))

    def w_cond(carry):
        k, _, changed = carry
        return jnp.logical_and(k < 30, changed)

    def w_body(carry):
        k, tau, _ = carry
        t2 = newton(tau)
        return k + 1, t2, t2 != tau

    _, tau, _ = lax.while_loop(w_cond, w_body, (0, tau, jnp.bool_(True)))

    def out_body(i, carry):
        sl = pl.ds(i * L, L)
        row_v[sl] = jnp.maximum(row_v[sl] - tau, 0.0)
        return carry

    lax.fori_loop(0, n_chunks, out_body, 0)


def _sc_sparsemax(x):
    rows, cols = x.shape
    n_chunks = cols // _SC_LANES
    rpw = rows // _SC_WORKERS
    mesh = plsc.VectorSubcoreMesh(core_axis_name="c", subcore_axis_name="s")
    cp = pltpu.CompilerParams()
    if "needs_layout_passes" in pltpu.CompilerParams.__dataclass_fields__:
        cp = dataclasses.replace(cp, needs_layout_passes=False)

    @functools.partial(
        pl.kernel,
        out_type=jax.ShapeDtypeStruct((rows, cols), x.dtype),
        mesh=mesh,
        compiler_params=cp,
        scratch_types=[
            pltpu.VMEM((cols,), jnp.float32),
            pltpu.SMEM((n_chunks,), jnp.int32),
        ],
    )
    def k(x_hbm, o_hbm, row_v, idx_ref):
        wid = lax.axis_index("s") * _SC_CORES + lax.axis_index("c")
        for r in range(rpw):
            row = wid * rpw + r
            pltpu.sync_copy(x_hbm.at[row], row_v)
            _sc_row_sparsemax(row_v, idx_ref, n_chunks)
            pltpu.sync_copy(row_v, o_hbm.at[row])

    return k(x)


@functools.partial(jax.jit, static_argnames=())
def kernel(x):
    return _tc_kernel(x)


def _tc_kernel(x):
    return pl.pallas_call(
        _sparsemax_block,
        out_shape=jax.ShapeDtypeStruct(x.shape, x.dtype),
    )(x)
